# baseline (speedup 1.0000x reference)
import functools

import jax
import jax.numpy as jnp
from jax import lax
from jax.experimental import pallas as pl
from jax.experimental.pallas import tpu as pltpu
from jax.experimental.pallas import tpu_sc as plsc


def kernel(input, indices):
    B, C, H, W = input.shape
    SPLIT = 8
    D = (H * W) // SPLIT
    ROWS = B * C * SPLIT

    info = plsc.get_sparse_core_info()
    NW = info.num_cores * info.num_subcores
    per_w = ROWS // NW
    K = 8
    chunks = per_w // K
    npairs = chunks // 2

    q = jnp.arange(ROWS, dtype=jnp.int32)
    coarse, sub = q // SPLIT, q % SPLIT
    b, c = coarse // C, coarse % C
    gidx = (b * C + indices[c]) * SPLIT + sub

    x2d = input.reshape(ROWS, D)
    mesh = plsc.VectorSubcoreMesh(core_axis_name="c", subcore_axis_name="s")

    @functools.partial(
        pl.kernel,
        out_type=jax.ShapeDtypeStruct((ROWS, D), jnp.float32),
        mesh=mesh,
        scratch_types=[
            pltpu.VMEM((per_w,), jnp.int32),
            pltpu.VMEM((K, D), jnp.float32),
            pltpu.VMEM((K, D), jnp.float32),
            pltpu.SemaphoreType.DMA,
            pltpu.SemaphoreType.DMA,
            pltpu.SemaphoreType.DMA,
            pltpu.SemaphoreType.DMA,
        ],
    )
    def run(in_hbm, gidx_hbm, out_hbm, idx_v, rows0, rows1, g0, g1, s0, s1):
        wid = lax.axis_index("s") * info.num_cores + lax.axis_index("c")
        base = wid * per_w
        pltpu.sync_copy(gidx_hbm.at[pl.ds(base, per_w)], idx_v)

        def gather(t, rows, sem):
            pltpu.make_async_copy(
                in_hbm.at[idx_v.at[pl.ds(t * K, K)]], rows, sem
            ).start()

        def gwait(rows, sem):
            pltpu.make_async_copy(in_hbm.at[pl.ds(0, K)], rows, sem).wait()

        def scatter(t, rows, sem):
            pltpu.make_async_copy(
                rows, out_hbm.at[pl.ds(base + t * K, K)], sem
            ).start()

        def swait(rows, sem):
            pltpu.make_async_copy(rows, out_hbm.at[pl.ds(base, K)], sem).wait()

        gather(0, rows0, g0)
        gather(1, rows1, g1)

        def pair(p, carry):
            t0 = 2 * p
            gwait(rows0, g0)
            scatter(t0, rows0, s0)
            gwait(rows1, g1)
            scatter(t0 + 1, rows1, s1)
            swait(rows0, s0)
            gather(t0 + 2, rows0, g0)
            swait(rows1, s1)
            gather(t0 + 3, rows1, g1)
            return carry

        lax.fori_loop(0, npairs - 1, pair, 0)

        t0 = 2 * (npairs - 1)
        gwait(rows0, g0)
        scatter(t0, rows0, s0)
        gwait(rows1, g1)
        scatter(t0 + 1, rows1, s1)
        swait(rows0, s0)
        swait(rows1, s1)

    out2d = run(x2d, gidx)
    return out2d.reshape(B, C, H, W)
